# Initial kernel scaffold; baseline (speedup 1.0000x reference)
import jax
import jax.numpy as jnp
from jax.experimental import pallas as pl


def kernel(in_feat, edge_index, W1, b1, W2, b2):
    raise NotImplementedError("write your pallas kernel here")


# speedup vs baseline: 7.4999x; 7.4999x over previous
import functools

import jax
import jax.numpy as jnp
from jax import lax
from jax.experimental import pallas as pl
from jax.experimental.pallas import tpu as pltpu
from jax.experimental.pallas import tpu_sc as plsc

N_NODES = 10000
NP = 10240
D = 128
E_EDGES = 320000
NCORE = 2
NSUB = 16
NW = NCORE * NSUB
EPW = E_EDGES // NW
B_EDGE = 80
NB = EPW // B_EDGE
RPW = NP // NSUB
ROWS_TC = 512
GRID_TC = NP // ROWS_TC

_mesh = plsc.VectorSubcoreMesh(core_axis_name="c", subcore_axis_name="s")


@functools.partial(
    pl.kernel,
    out_type=jax.ShapeDtypeStruct((NCORE, 2, NP), jnp.float32),
    mesh=_mesh,
    scratch_types=[
        pltpu.VMEM((B_EDGE,), jnp.int32),
        pltpu.VMEM((B_EDGE,), jnp.int32),
        pltpu.VMEM((B_EDGE,), jnp.float32),
        pltpu.VMEM_SHARED((NP,), jnp.float32),
        pltpu.VMEM_SHARED((NP,), jnp.float32),
        pltpu.SemaphoreType.DMA,
    ],
)
def _deg_kernel(src_hbm, dst_hbm, z1_hbm, out_hbm, src_v, dst_v, ones_v,
                hsrc, hdst, sem):
    c = lax.axis_index("c")
    s = lax.axis_index("s")
    wid = c * NSUB + s
    pltpu.sync_copy(z1_hbm, hsrc.at[pl.ds(s * RPW, RPW)])
    pltpu.sync_copy(z1_hbm, hdst.at[pl.ds(s * RPW, RPW)])
    for i in range(B_EDGE // 16):
        ones_v[pl.ds(i * 16, 16)] = jnp.ones((16,), jnp.float32)
    plsc.subcore_barrier()
    ebase = wid * EPW

    def body(j, carry):
        b = ebase + j * B_EDGE
        pltpu.sync_copy(src_hbm.at[pl.ds(b, B_EDGE)], src_v)
        pltpu.sync_copy(dst_hbm.at[pl.ds(b, B_EDGE)], dst_v)
        pltpu.sync_copy(ones_v, hsrc.at[src_v], add=True)
        pltpu.sync_copy(ones_v, hdst.at[dst_v], add=True)
        return carry

    lax.fori_loop(0, NB, body, 0)
    plsc.subcore_barrier()
    pltpu.sync_copy(hsrc.at[pl.ds(s * RPW, RPW)], out_hbm.at[c, 0, pl.ds(s * RPW, RPW)])
    pltpu.sync_copy(hdst.at[pl.ds(s * RPW, RPW)], out_hbm.at[c, 1, pl.ds(s * RPW, RPW)])


@functools.partial(
    pl.kernel,
    out_type=jax.ShapeDtypeStruct((NCORE, NP, D), jnp.float32),
    mesh=_mesh,
    scratch_types=[
        pltpu.VMEM((B_EDGE,), jnp.int32),
        pltpu.VMEM((B_EDGE,), jnp.int32),
        pltpu.VMEM((B_EDGE, D), jnp.float32),
        pltpu.VMEM_SHARED((NP, D), jnp.float32),
        pltpu.SemaphoreType.DMA,
    ],
)
def _agg_kernel(y_hbm, src_hbm, dst_hbm, z2_hbm, out_hbm, src_v, dst_v,
                rows_v, acc, sem):
    c = lax.axis_index("c")
    s = lax.axis_index("s")
    wid = c * NSUB + s
    pltpu.sync_copy(z2_hbm, acc.at[pl.ds(s * RPW, RPW)])
    plsc.subcore_barrier()
    ebase = wid * EPW

    def body(j, carry):
        b = ebase + j * B_EDGE
        pltpu.sync_copy(src_hbm.at[pl.ds(b, B_EDGE)], src_v)
        pltpu.sync_copy(dst_hbm.at[pl.ds(b, B_EDGE)], dst_v)
        pltpu.async_copy(y_hbm.at[src_v], rows_v, sem).wait()
        pltpu.sync_copy(rows_v, acc.at[dst_v], add=True)
        return carry

    lax.fori_loop(0, NB, body, 0)
    plsc.subcore_barrier()
    pltpu.sync_copy(acc.at[pl.ds(s * RPW, RPW)], out_hbm.at[c, pl.ds(s * RPW, RPW)])


def _mm_scale_body(x_ref, w_ref, dsrc_ref, y_ref):
    deg = dsrc_ref[0, :] + dsrc_ref[1, :]
    ns = lax.rsqrt(jnp.maximum(deg, 1.0))
    y_ref[...] = jnp.dot(x_ref[...], w_ref[...],
                         preferred_element_type=jnp.float32) * ns[:, None]


_mm_scale = pl.pallas_call(
    _mm_scale_body,
    grid=(GRID_TC,),
    in_specs=[
        pl.BlockSpec((ROWS_TC, D), lambda i: (i, 0)),
        pl.BlockSpec((D, D), lambda i: (0, 0)),
        pl.BlockSpec((NCORE, ROWS_TC), lambda i: (0, i)),
    ],
    out_specs=pl.BlockSpec((ROWS_TC, D), lambda i: (i, 0)),
    out_shape=jax.ShapeDtypeStruct((NP, D), jnp.float32),
)


def _comb_mm_body(p_ref, ddst_ref, b_ref, dsrc_ref, w_ref, y_ref):
    nd = lax.rsqrt(jnp.maximum(ddst_ref[0, :] + ddst_ref[1, :], 1.0))
    h = jnp.maximum((p_ref[0] + p_ref[1]) * nd[:, None] + b_ref[...], 0.0)
    ns = lax.rsqrt(jnp.maximum(dsrc_ref[0, :] + dsrc_ref[1, :], 1.0))
    y_ref[...] = jnp.dot(h, w_ref[...],
                         preferred_element_type=jnp.float32) * ns[:, None]


_comb_mm = pl.pallas_call(
    _comb_mm_body,
    grid=(GRID_TC,),
    in_specs=[
        pl.BlockSpec((NCORE, ROWS_TC, D), lambda i: (0, i, 0)),
        pl.BlockSpec((NCORE, ROWS_TC), lambda i: (0, i)),
        pl.BlockSpec((1, D), lambda i: (0, 0)),
        pl.BlockSpec((NCORE, ROWS_TC), lambda i: (0, i)),
        pl.BlockSpec((D, D), lambda i: (0, 0)),
    ],
    out_specs=pl.BlockSpec((ROWS_TC, D), lambda i: (i, 0)),
    out_shape=jax.ShapeDtypeStruct((NP, D), jnp.float32),
)


def _final_body(p_ref, ddst_ref, b_ref, out_ref):
    nd = lax.rsqrt(jnp.maximum(ddst_ref[0, :] + ddst_ref[1, :], 1.0))
    z = jnp.maximum((p_ref[0] + p_ref[1]) * nd[:, None] + b_ref[...], 0.0)
    z = z - jnp.max(z, axis=1, keepdims=True)
    e = jnp.exp(z)
    out_ref[...] = e / jnp.sum(e, axis=1, keepdims=True)


_final = pl.pallas_call(
    _final_body,
    grid=(GRID_TC,),
    in_specs=[
        pl.BlockSpec((NCORE, ROWS_TC, D), lambda i: (0, i, 0)),
        pl.BlockSpec((NCORE, ROWS_TC), lambda i: (0, i)),
        pl.BlockSpec((1, D), lambda i: (0, 0)),
    ],
    out_specs=pl.BlockSpec((ROWS_TC, D), lambda i: (i, 0)),
    out_shape=jax.ShapeDtypeStruct((NP, D), jnp.float32),
)


def kernel(in_feat, edge_index, W1, b1, W2, b2):
    src = edge_index[0]
    dst = edge_index[1]
    z1 = jnp.zeros((RPW,), jnp.float32)
    z2 = jnp.zeros((RPW, D), jnp.float32)
    degp = _deg_kernel(src, dst, z1)
    dsrc = degp[:, 0, :]
    ddst = degp[:, 1, :]
    b1r = b1.reshape(1, D)
    b2r = b2.reshape(1, D)
    y1 = _mm_scale(in_feat, W1, dsrc)
    p1 = _agg_kernel(y1, src, dst, z2)
    y2 = _comb_mm(p1, ddst, b1r, dsrc, W2)
    p2 = _agg_kernel(y2, src, dst, z2)
    out = _final(p2, ddst, b2r)
    return out[:N_NODES, :]
